# Initial kernel scaffold; baseline (speedup 1.0000x reference)
import jax
import jax.numpy as jnp
from jax.experimental import pallas as pl


def kernel(graph_sig, edge_index, W1, b1, W2, b2, W3, b3, W4, b4):
    raise NotImplementedError("write your pallas kernel here")


# speedup vs baseline: 3.0746x; 3.0746x over previous
import functools

import jax
import jax.numpy as jnp
from jax import lax
from jax.experimental import pallas as pl
from jax.experimental.pallas import tpu as pltpu
from jax.experimental.pallas import tpu_sc as plsc

N = 10000
C = 128
H = 512
NPAD = 10240
NC = 2
NS = 16
NW = NC * NS
CHUNK = 128
K = 80
EPW = K * CHUNK
EPAD = NW * EPW
ROWS_PER_TILE = NPAD // NS


def _sc_partial_agg(x_pad, src_idx, dst_idx, zeros_pad):
    mesh = plsc.VectorSubcoreMesh(core_axis_name="c", subcore_axis_name="s")

    @functools.partial(
        pl.kernel,
        out_type=jax.ShapeDtypeStruct((NC, NPAD, C), jnp.float32),
        mesh=mesh,
        scratch_types=[
            pltpu.VMEM((K, CHUNK), jnp.int32),
            pltpu.VMEM((K, CHUNK), jnp.int32),
            pltpu.VMEM((CHUNK, C), jnp.float32),
            pltpu.VMEM_SHARED((NPAD, C), jnp.float32),
            pltpu.SemaphoreType.DMA,
        ],
    )
    def sc_agg(x_hbm, src_hbm, dst_hbm, zero_hbm, out_hbm,
               src_v, dst_v, rows_v, acc_sh, sem0):
        cid = lax.axis_index("c")
        sid = lax.axis_index("s")
        wid = cid * NS + sid

        pltpu.sync_copy(src_hbm.at[wid], src_v)
        pltpu.sync_copy(dst_hbm.at[wid], dst_v)
        row0 = sid * ROWS_PER_TILE
        pltpu.sync_copy(zero_hbm.at[pl.ds(row0, ROWS_PER_TILE)],
                        acc_sh.at[pl.ds(row0, ROWS_PER_TILE)])
        plsc.subcore_barrier()

        def body(j, carry):
            pltpu.make_async_copy(
                x_hbm.at[src_v.at[j]], rows_v, sem0).start()
            pltpu.make_async_copy(
                x_hbm.at[src_v.at[j]], rows_v, sem0).wait()
            pltpu.sync_copy(rows_v, acc_sh.at[dst_v.at[j]], add=True)
            return carry

        lax.fori_loop(0, K, body, 0)
        plsc.subcore_barrier()
        pltpu.sync_copy(acc_sh.at[pl.ds(row0, ROWS_PER_TILE)],
                        out_hbm.at[cid, pl.ds(row0, ROWS_PER_TILE)])

    return sc_agg(x_pad, src_idx, dst_idx, zeros_pad)


def _mlp(x_pad, a0, a1, Wa, ba, Wb, bb):
    BN = 512

    def body(x_ref, a0_ref, a1_ref, wa_ref, ba_ref, wb_ref, bb_ref, o_ref):
        h = x_ref[...] + a0_ref[...] + a1_ref[...]
        z = jnp.dot(h, wa_ref[...], preferred_element_type=jnp.float32)
        z = jnp.maximum(z + ba_ref[...], 0.0)
        y = jnp.dot(z, wb_ref[...], preferred_element_type=jnp.float32)
        y = y + bb_ref[...]
        rows = pl.program_id(0) * BN + lax.broadcasted_iota(
            jnp.int32, (BN, 1), 0)
        o_ref[...] = jnp.where(rows < N, y, 0.0)

    return pl.pallas_call(
        body,
        grid=(NPAD // BN,),
        in_specs=[
            pl.BlockSpec((BN, C), lambda i: (i, 0)),
            pl.BlockSpec((BN, C), lambda i: (i, 0)),
            pl.BlockSpec((BN, C), lambda i: (i, 0)),
            pl.BlockSpec((C, H), lambda i: (0, 0)),
            pl.BlockSpec((1, H), lambda i: (0, 0)),
            pl.BlockSpec((H, C), lambda i: (0, 0)),
            pl.BlockSpec((1, C), lambda i: (0, 0)),
        ],
        out_specs=pl.BlockSpec((BN, C), lambda i: (i, 0)),
        out_shape=jax.ShapeDtypeStruct((NPAD, C), jnp.float32),
    )(x_pad, a0, a1, Wa, ba.reshape(1, H), Wb, bb.reshape(1, C))


def kernel(graph_sig, edge_index, W1, b1, W2, b2, W3, b3, W4, b4):
    x0 = graph_sig[0].astype(jnp.float32)
    x_pad = jnp.zeros((NPAD, C), jnp.float32).at[:N].set(x0)

    E = edge_index.shape[1]
    ei = edge_index.astype(jnp.int32)
    fill = jnp.full((EPAD - E,), N, jnp.int32)
    src_idx = jnp.concatenate([ei[0], fill]).reshape(NW, K, CHUNK)
    dst_idx = jnp.concatenate([ei[1], fill]).reshape(NW, K, CHUNK)
    zeros_pad = jnp.zeros((NPAD, C), jnp.float32)

    agg = _sc_partial_agg(x_pad, src_idx, dst_idx, zeros_pad)
    y1 = _mlp(x_pad, agg[0], agg[1], W1, b1, W2, b2)
    agg2 = _sc_partial_agg(y1, src_idx, dst_idx, zeros_pad)
    y2 = _mlp(y1, agg2[0], agg2[1], W3, b3, W4, b4)
    return y2[:N][None]


# speedup vs baseline: 3.5762x; 1.1631x over previous
import functools

import jax
import jax.numpy as jnp
from jax import lax
from jax.experimental import pallas as pl
from jax.experimental.pallas import tpu as pltpu
from jax.experimental.pallas import tpu_sc as plsc

N = 10000
C = 128
H = 512
NPAD = 10240
NC = 2
NS = 16
NW = NC * NS
CHUNK = 128
K = 80
EPW = K * CHUNK
EPAD = NW * EPW
ROWS_PER_TILE = NPAD // NS


def _sc_partial_agg(x_pad, src_idx, dst_idx, zeros_pad):
    mesh = plsc.VectorSubcoreMesh(core_axis_name="c", subcore_axis_name="s")

    @functools.partial(
        pl.kernel,
        out_type=jax.ShapeDtypeStruct((NC, NPAD, C), jnp.float32),
        mesh=mesh,
        scratch_types=[
            pltpu.VMEM((K, CHUNK), jnp.int32),
            pltpu.VMEM((2, CHUNK), jnp.int32),
            pltpu.VMEM((2, CHUNK, C), jnp.float32),
            pltpu.VMEM_SHARED((NPAD, C), jnp.float32),
            pltpu.SemaphoreType.DMA,
            pltpu.SemaphoreType.DMA,
            pltpu.SemaphoreType.DMA,
            pltpu.SemaphoreType.DMA,
        ],
    )
    def sc_agg(x_hbm, src_hbm, dst_hbm, zero_hbm, out_hbm,
               src_v, dst_v, rows_v, acc_sh, gsem0, gsem1, dsem0, dsem1):
        cid = lax.axis_index("c")
        sid = lax.axis_index("s")
        wid = cid * NS + sid
        gsems = (gsem0, gsem1)
        dsems = (dsem0, dsem1)

        pltpu.sync_copy(src_hbm.at[wid], src_v)
        row0 = sid * ROWS_PER_TILE
        pltpu.sync_copy(zero_hbm.at[pl.ds(row0, ROWS_PER_TILE)],
                        acc_sh.at[pl.ds(row0, ROWS_PER_TILE)])
        plsc.subcore_barrier()

        def gather(j, b):
            return pltpu.make_async_copy(
                x_hbm.at[src_v.at[j]], rows_v.at[b], gsems[b])

        def dcp(j, b):
            return pltpu.make_async_copy(
                dst_hbm.at[wid, j], dst_v.at[b], dsems[b])

        gather(0, 0).start()
        dcp(0, 0).start()
        gather(1, 1).start()
        dcp(1, 1).start()

        def body(j2, carry):
            for b in range(2):
                ch = j2 * 2 + b
                gather(ch, b).wait()
                dcp(ch, b).wait()
                pltpu.sync_copy(rows_v.at[b], acc_sh.at[dst_v.at[b]],
                                add=True)

                @pl.when(ch + 2 < K)
                def _():
                    gather(ch + 2, b).start()
                    dcp(ch + 2, b).start()
            return carry

        lax.fori_loop(0, K // 2, body, 0)
        plsc.subcore_barrier()
        pltpu.sync_copy(acc_sh.at[pl.ds(row0, ROWS_PER_TILE)],
                        out_hbm.at[cid, pl.ds(row0, ROWS_PER_TILE)])

    return sc_agg(x_pad, src_idx, dst_idx, zeros_pad)


def _mlp(x_pad, a0, a1, Wa, ba, Wb, bb):
    BN = 512

    def body(x_ref, a0_ref, a1_ref, wa_ref, ba_ref, wb_ref, bb_ref, o_ref):
        h = x_ref[...] + a0_ref[...] + a1_ref[...]
        z = jnp.dot(h, wa_ref[...], preferred_element_type=jnp.float32)
        z = jnp.maximum(z + ba_ref[...], 0.0)
        y = jnp.dot(z, wb_ref[...], preferred_element_type=jnp.float32)
        y = y + bb_ref[...]
        rows = pl.program_id(0) * BN + lax.broadcasted_iota(
            jnp.int32, (BN, 1), 0)
        o_ref[...] = jnp.where(rows < N, y, 0.0)

    return pl.pallas_call(
        body,
        grid=(NPAD // BN,),
        in_specs=[
            pl.BlockSpec((BN, C), lambda i: (i, 0)),
            pl.BlockSpec((BN, C), lambda i: (i, 0)),
            pl.BlockSpec((BN, C), lambda i: (i, 0)),
            pl.BlockSpec((C, H), lambda i: (0, 0)),
            pl.BlockSpec((1, H), lambda i: (0, 0)),
            pl.BlockSpec((H, C), lambda i: (0, 0)),
            pl.BlockSpec((1, C), lambda i: (0, 0)),
        ],
        out_specs=pl.BlockSpec((BN, C), lambda i: (i, 0)),
        out_shape=jax.ShapeDtypeStruct((NPAD, C), jnp.float32),
    )(x_pad, a0, a1, Wa, ba.reshape(1, H), Wb, bb.reshape(1, C))


def kernel(graph_sig, edge_index, W1, b1, W2, b2, W3, b3, W4, b4):
    x0 = graph_sig[0].astype(jnp.float32)
    x_pad = jnp.zeros((NPAD, C), jnp.float32).at[:N].set(x0)

    E = edge_index.shape[1]
    ei = edge_index.astype(jnp.int32)
    fill = jnp.full((EPAD - E,), N, jnp.int32)
    src_idx = jnp.concatenate([ei[0], fill]).reshape(NW, K, CHUNK)
    dst_idx = jnp.concatenate([ei[1], fill]).reshape(NW, K, CHUNK)
    zeros_pad = jnp.zeros((NPAD, C), jnp.float32)

    agg = _sc_partial_agg(x_pad, src_idx, dst_idx, zeros_pad)
    y1 = _mlp(x_pad, agg[0], agg[1], W1, b1, W2, b2)
    agg2 = _sc_partial_agg(y1, src_idx, dst_idx, zeros_pad)
    y2 = _mlp(y1, agg2[0], agg2[1], W3, b3, W4, b4)
    return y2[:N][None]
